# baseline (speedup 1.0000x reference)
import functools

import jax
import jax.numpy as jnp
from jax import lax
from jax.experimental import pallas as pl
from jax.experimental.pallas import tpu as pltpu
from jax.experimental.pallas import tpu_sc as plsc

_N = 10000
_E = 320000
_D = 128

_NP = 10240
_ROWS_PER_TILE = _NP // 16
_CHUNK = 64
_NW = 32
_CPW = 160
_EP = _NW * _CPW * _CHUNK

_mesh = plsc.VectorSubcoreMesh(core_axis_name="c", subcore_axis_name="s")


@functools.partial(
    pl.kernel,
    out_type=jax.ShapeDtypeStruct((2, _NP, _D), jnp.float32),
    mesh=_mesh,
    scratch_types=[
        pltpu.VMEM_SHARED((_NP, _D), jnp.float32),
        pltpu.VMEM((_CPW // 4, _CHUNK), jnp.int32),
        pltpu.VMEM((_CPW // 4, _CHUNK), jnp.int32),
        [pltpu.VMEM((_CHUNK, _D), jnp.float32) for _ in range(4)],
        [pltpu.SemaphoreType.DMA for _ in range(4)],
        [pltpu.SemaphoreType.DMA for _ in range(4)],
    ],
)
def _sc_agg(h_hbm, src_hbm, dst_hbm, zeros_hbm, out_hbm,
            agg_sh, src_v, dst_v, rows, gsem, ssem):
    c = lax.axis_index("c")
    s = lax.axis_index("s")
    wid = c * 16 + s

    row0 = s * _ROWS_PER_TILE
    pltpu.sync_copy(zeros_hbm.at[pl.ds(row0, _ROWS_PER_TILE)],
                    agg_sh.at[pl.ds(row0, _ROWS_PER_TILE)])

    plsc.subcore_barrier()

    half = _CPW // 4
    for p in range(4):
        base = wid * _CPW + p * half
        pltpu.sync_copy(src_hbm.at[pl.ds(base, half)], src_v)
        pltpu.sync_copy(dst_hbm.at[pl.ds(base, half)], dst_v)
        pltpu.async_copy(h_hbm.at[src_v.at[0]], rows[0], gsem[0])
        pltpu.async_copy(h_hbm.at[src_v.at[1]], rows[1], gsem[1])

        @pl.loop(0, half, step=4)
        def _(k0):
            for b in range(4):
                k = k0 + b
                b2 = (b + 2) % 4

                @pl.when(k >= 2)
                def _():
                    pltpu.make_async_copy(
                        rows[b2], agg_sh.at[dst_v.at[k - 2]], ssem[b2]).wait()

                @pl.when(k + 2 < half)
                def _():
                    pltpu.async_copy(h_hbm.at[src_v.at[k + 2]], rows[b2],
                                     gsem[b2])

                pltpu.make_async_copy(h_hbm.at[src_v.at[k]], rows[b],
                                      gsem[b]).wait()
                pltpu.async_copy(rows[b], agg_sh.at[dst_v.at[k]], ssem[b],
                                 add=True)

        for k in (half - 2, half - 1):
            pltpu.make_async_copy(rows[k % 4], agg_sh.at[dst_v.at[k]],
                                  ssem[k % 4]).wait()

    plsc.subcore_barrier()

    pltpu.sync_copy(agg_sh.at[pl.ds(row0, _ROWS_PER_TILE)],
                    out_hbm.at[c, pl.ds(row0, _ROWS_PER_TILE)])


def _mlp_body(h_ref, a0_ref, a1_ref, w1_ref, b1_ref, w2_ref, b2_ref, o_ref):
    t = h_ref[...] + a0_ref[...] + a1_ref[...]
    t = jnp.dot(t, w1_ref[...], preferred_element_type=jnp.float32) + b1_ref[...]
    t = jnp.maximum(t, 0.0)
    o_ref[...] = (
        jnp.dot(t, w2_ref[...], preferred_element_type=jnp.float32) + b2_ref[...]
    )


_ROW_BLK = 1000


def _mlp(h, a0, a1, w1, b1, w2, b2):
    grid = (_N // _ROW_BLK,)
    blk = lambda i: (i, 0)
    fixed = lambda i: (0, 0)
    return pl.pallas_call(
        _mlp_body,
        grid=grid,
        in_specs=[
            pl.BlockSpec((_ROW_BLK, _D), blk),
            pl.BlockSpec((_ROW_BLK, _D), blk),
            pl.BlockSpec((_ROW_BLK, _D), blk),
            pl.BlockSpec((_D, _D), fixed),
            pl.BlockSpec((1, _D), fixed),
            pl.BlockSpec((_D, _D), fixed),
            pl.BlockSpec((1, _D), fixed),
        ],
        out_specs=pl.BlockSpec((_ROW_BLK, _D), blk),
        out_shape=jax.ShapeDtypeStruct((_N, _D), jnp.float32),
    )(h, a0, a1, w1, b1, w2, b2)


def kernel(x, edge_index, W1_0, b1_0, W2_0, b2_0, W1_1, b1_1, W2_1, b2_1,
           W1_2, b1_2, W2_2, b2_2):
    params = [
        (W1_0, b1_0, W2_0, b2_0),
        (W1_1, b1_1, W2_1, b2_1),
        (W1_2, b1_2, W2_2, b2_2),
    ]
    npad = _EP - _E
    pad_iota = jnp.arange(npad, dtype=jnp.int32)
    src2d = jnp.concatenate(
        [edge_index[0], pad_iota % _N]).reshape(-1, _CHUNK)
    dst2d = jnp.concatenate(
        [edge_index[1], _N + pad_iota % (_NP - _N)]).reshape(-1, _CHUNK)
    zeros = jnp.zeros((_NP, _D), jnp.float32)
    h = x
    for (W1, b1, W2, b2) in params:
        agg = _sc_agg(h, src2d, dst2d, zeros)
        h = _mlp(h, agg[0][:_N], agg[1][:_N], W1, b1.reshape(1, _D), W2,
                 b2.reshape(1, _D))
    return h


# speedup vs baseline: 1.0400x; 1.0400x over previous
import functools

import jax
import jax.numpy as jnp
from jax import lax
from jax.experimental import pallas as pl
from jax.experimental.pallas import tpu as pltpu
from jax.experimental.pallas import tpu_sc as plsc

_N = 10000
_E = 320000
_D = 128

_NP = 10240
_ROWS_PER_TILE = _NP // 16
_CHUNK = 128
_NW = 32
_CPW = 80
_EP = _NW * _CPW * _CHUNK

_mesh = plsc.VectorSubcoreMesh(core_axis_name="c", subcore_axis_name="s")


@functools.partial(
    pl.kernel,
    out_type=jax.ShapeDtypeStruct((2, _NP, _D), jnp.float32),
    mesh=_mesh,
    scratch_types=[
        pltpu.VMEM_SHARED((_NP, _D), jnp.float32),
        pltpu.VMEM((_CPW // 2, _CHUNK), jnp.int32),
        pltpu.VMEM((_CPW // 2, _CHUNK), jnp.int32),
        pltpu.VMEM((_CHUNK, _D), jnp.float32),
        pltpu.VMEM((_CHUNK, _D), jnp.float32),
        pltpu.SemaphoreType.DMA,
        pltpu.SemaphoreType.DMA,
    ],
)
def _sc_agg(h_hbm, src_hbm, dst_hbm, zeros_hbm, out_hbm,
            agg_sh, src_v, dst_v, rows0, rows1, sem0, sem1):
    c = lax.axis_index("c")
    s = lax.axis_index("s")
    wid = c * 16 + s

    row0 = s * _ROWS_PER_TILE

    @pl.when(jnp.logical_and(c == 0, s < 15))
    def _():
        pltpu.sync_copy(h_hbm.at[pl.ds(row0, _ROWS_PER_TILE)],
                        agg_sh.at[pl.ds(row0, _ROWS_PER_TILE)])

    @pl.when(jnp.logical_and(c == 0, s == 15))
    def _():
        pltpu.sync_copy(h_hbm.at[pl.ds(row0, _N - 15 * _ROWS_PER_TILE)],
                        agg_sh.at[pl.ds(row0, _N - 15 * _ROWS_PER_TILE)])
        pltpu.sync_copy(zeros_hbm.at[pl.ds(_N, _NP - _N)],
                        agg_sh.at[pl.ds(_N, _NP - _N)])

    @pl.when(c == 1)
    def _():
        pltpu.sync_copy(zeros_hbm.at[pl.ds(row0, _ROWS_PER_TILE)],
                        agg_sh.at[pl.ds(row0, _ROWS_PER_TILE)])

    half = _CPW // 2
    for p in range(2):
        base = wid * _CPW + p * half
        pltpu.sync_copy(src_hbm.at[pl.ds(base, half)], src_v)
        pltpu.sync_copy(dst_hbm.at[pl.ds(base, half)], dst_v)
        pltpu.async_copy(h_hbm.at[src_v.at[0]], rows0, sem0)
        if p == 0:
            plsc.subcore_barrier()

        @pl.loop(0, half, step=2)
        def _(j):
            pltpu.async_copy(h_hbm.at[src_v.at[j + 1]], rows1, sem1)
            pltpu.make_async_copy(h_hbm.at[src_v.at[j]], rows0, sem0).wait()
            pltpu.sync_copy(rows0, agg_sh.at[dst_v.at[j]], add=True)

            @pl.when(j + 2 < half)
            def _():
                pltpu.async_copy(h_hbm.at[src_v.at[j + 2]], rows0, sem0)

            pltpu.make_async_copy(h_hbm.at[src_v.at[j + 1]], rows1, sem1).wait()
            pltpu.sync_copy(rows1, agg_sh.at[dst_v.at[j + 1]], add=True)

    plsc.subcore_barrier()

    pltpu.sync_copy(agg_sh.at[pl.ds(row0, _ROWS_PER_TILE)],
                    out_hbm.at[c, pl.ds(row0, _ROWS_PER_TILE)])


def _mlp_body(a0_ref, a1_ref, w1_ref, b1_ref, w2_ref, b2_ref, o_ref):
    t = a0_ref[...] + a1_ref[...]
    t = jnp.dot(t, w1_ref[...], preferred_element_type=jnp.float32) + b1_ref[...]
    t = jnp.maximum(t, 0.0)
    o_ref[...] = (
        jnp.dot(t, w2_ref[...], preferred_element_type=jnp.float32) + b2_ref[...]
    )


_ROW_BLK = 1000


def _mlp(a0, a1, w1, b1, w2, b2):
    grid = (_N // _ROW_BLK,)
    blk = lambda i: (i, 0)
    fixed = lambda i: (0, 0)
    return pl.pallas_call(
        _mlp_body,
        grid=grid,
        in_specs=[
            pl.BlockSpec((_ROW_BLK, _D), blk),
            pl.BlockSpec((_ROW_BLK, _D), blk),
            pl.BlockSpec((_D, _D), fixed),
            pl.BlockSpec((1, _D), fixed),
            pl.BlockSpec((_D, _D), fixed),
            pl.BlockSpec((1, _D), fixed),
        ],
        out_specs=pl.BlockSpec((_ROW_BLK, _D), blk),
        out_shape=jax.ShapeDtypeStruct((_N, _D), jnp.float32),
    )(a0, a1, w1, b1, w2, b2)


def kernel(x, edge_index, W1_0, b1_0, W2_0, b2_0, W1_1, b1_1, W2_1, b2_1,
           W1_2, b1_2, W2_2, b2_2):
    params = [
        (W1_0, b1_0, W2_0, b2_0),
        (W1_1, b1_1, W2_1, b2_1),
        (W1_2, b1_2, W2_2, b2_2),
    ]
    npad = _EP - _E
    pad_iota = jnp.arange(npad, dtype=jnp.int32)
    src2d = jnp.concatenate(
        [edge_index[0], pad_iota % _N]).reshape(-1, _CHUNK)
    dst2d = jnp.concatenate(
        [edge_index[1], _N + pad_iota % (_NP - _N)]).reshape(-1, _CHUNK)
    zeros = jnp.zeros((_NP, _D), jnp.float32)
    h = x
    for (W1, b1, W2, b2) in params:
        agg = _sc_agg(h, src2d, dst2d, zeros)
        h = _mlp(agg[0][:_N], agg[1][:_N], W1, b1.reshape(1, _D), W2,
                 b2.reshape(1, _D))
    return h


# speedup vs baseline: 1.1170x; 1.0740x over previous
import functools

import jax
import jax.numpy as jnp
from jax import lax
from jax.experimental import pallas as pl
from jax.experimental.pallas import tpu as pltpu
from jax.experimental.pallas import tpu_sc as plsc

_N = 10000
_E = 320000
_D = 128

_NP = 10240
_ROWS_PER_TILE = _NP // 16
_CHUNK = 128
_NW = 32
_CPW = 80
_EP = _NW * _CPW * _CHUNK

_mesh = plsc.VectorSubcoreMesh(core_axis_name="c", subcore_axis_name="s")


@functools.partial(
    pl.kernel,
    out_type=jax.ShapeDtypeStruct((2, _NP, _D), jnp.float32),
    mesh=_mesh,
    scratch_types=[
        pltpu.VMEM_SHARED((_NP, _D), jnp.float32),
        pltpu.VMEM((_CPW // 2, _CHUNK), jnp.int32),
        pltpu.VMEM((_CPW // 2, _CHUNK), jnp.int32),
        pltpu.VMEM((_CHUNK, _D), jnp.float32),
        pltpu.VMEM((_CHUNK, _D), jnp.float32),
        pltpu.SemaphoreType.DMA,
        pltpu.SemaphoreType.DMA,
    ],
)
def _sc_agg(h_hbm, src_hbm, dst_hbm, zeros_hbm, out_hbm,
            agg_sh, src_v, dst_v, rows0, rows1, sem0, sem1):
    c = lax.axis_index("c")
    s = lax.axis_index("s")
    wid = c * 16 + s

    row0 = s * _ROWS_PER_TILE

    @pl.when(jnp.logical_and(c == 0, s < 15))
    def _():
        pltpu.sync_copy(h_hbm.at[pl.ds(row0, _ROWS_PER_TILE)],
                        agg_sh.at[pl.ds(row0, _ROWS_PER_TILE)])

    @pl.when(jnp.logical_and(c == 0, s == 15))
    def _():
        pltpu.sync_copy(h_hbm.at[pl.ds(row0, _N - 15 * _ROWS_PER_TILE)],
                        agg_sh.at[pl.ds(row0, _N - 15 * _ROWS_PER_TILE)])
        pltpu.sync_copy(zeros_hbm.at[pl.ds(_N, _NP - _N)],
                        agg_sh.at[pl.ds(_N, _NP - _N)])

    @pl.when(c == 1)
    def _():
        pltpu.sync_copy(zeros_hbm.at[pl.ds(row0, _ROWS_PER_TILE)],
                        agg_sh.at[pl.ds(row0, _ROWS_PER_TILE)])

    half = _CPW // 2
    for p in range(2):
        base = wid * _CPW + p * half
        pltpu.sync_copy(src_hbm.at[pl.ds(base, half)], src_v)
        pltpu.sync_copy(dst_hbm.at[pl.ds(base, half)], dst_v)
        pltpu.async_copy(h_hbm.at[src_v.at[0]], rows0, sem0)
        if p == 0:
            plsc.subcore_barrier()

        @pl.loop(0, half, step=2)
        def _(j):
            pltpu.async_copy(h_hbm.at[src_v.at[j + 1]], rows1, sem1)
            pltpu.make_async_copy(h_hbm.at[src_v.at[j]], rows0, sem0).wait()
            pltpu.sync_copy(rows0, agg_sh.at[dst_v.at[j]], add=True)

            @pl.when(j + 2 < half)
            def _():
                pltpu.async_copy(h_hbm.at[src_v.at[j + 2]], rows0, sem0)

            pltpu.make_async_copy(h_hbm.at[src_v.at[j + 1]], rows1, sem1).wait()
            pltpu.sync_copy(rows1, agg_sh.at[dst_v.at[j + 1]], add=True)

    plsc.subcore_barrier()

    pltpu.sync_copy(agg_sh.at[pl.ds(row0, _ROWS_PER_TILE)],
                    out_hbm.at[c, pl.ds(row0, _ROWS_PER_TILE)])


def _mlp_body(a_ref, w1_ref, b1_ref, w2_ref, b2_ref, o_ref):
    t = a_ref[0] + a_ref[1]
    t = jnp.dot(t, w1_ref[...], preferred_element_type=jnp.float32) + b1_ref[...]
    t = jnp.maximum(t, 0.0)
    o_ref[...] = (
        jnp.dot(t, w2_ref[...], preferred_element_type=jnp.float32) + b2_ref[...]
    )


_ROW_BLK = 2000


def _mlp(agg, w1, b1, w2, b2):
    grid = (_N // _ROW_BLK,)
    fixed = lambda i: (0, 0)
    return pl.pallas_call(
        _mlp_body,
        grid=grid,
        in_specs=[
            pl.BlockSpec((2, _ROW_BLK, _D), lambda i: (0, i, 0)),
            pl.BlockSpec((_D, _D), fixed),
            pl.BlockSpec((1, _D), fixed),
            pl.BlockSpec((_D, _D), fixed),
            pl.BlockSpec((1, _D), fixed),
        ],
        out_specs=pl.BlockSpec((_ROW_BLK, _D), lambda i: (i, 0)),
        out_shape=jax.ShapeDtypeStruct((_N, _D), jnp.float32),
    )(agg, w1, b1, w2, b2)


def kernel(x, edge_index, W1_0, b1_0, W2_0, b2_0, W1_1, b1_1, W2_1, b2_1,
           W1_2, b1_2, W2_2, b2_2):
    params = [
        (W1_0, b1_0, W2_0, b2_0),
        (W1_1, b1_1, W2_1, b2_1),
        (W1_2, b1_2, W2_2, b2_2),
    ]
    npad = _EP - _E
    pad_iota = jnp.arange(npad, dtype=jnp.int32)
    src2d = jnp.concatenate(
        [edge_index[0], pad_iota % _N]).reshape(-1, _CHUNK)
    dst2d = jnp.concatenate(
        [edge_index[1], _N + pad_iota % (_NP - _N)]).reshape(-1, _CHUNK)
    zeros = jnp.zeros((_NP, _D), jnp.float32)
    h = x
    for (W1, b1, W2, b2) in params:
        agg = _sc_agg(h, src2d, dst2d, zeros)
        h = _mlp(agg, W1, b1.reshape(1, _D), W2, b2.reshape(1, _D))
    return h


# speedup vs baseline: 1.1449x; 1.0250x over previous
import functools

import jax
import jax.numpy as jnp
from jax import lax
from jax.experimental import pallas as pl
from jax.experimental.pallas import tpu as pltpu
from jax.experimental.pallas import tpu_sc as plsc

_N = 10000
_E = 320000
_D = 128

_NP = 10240
_ROWS_PER_TILE = _NP // 16
_CHUNK = 128
_NW = 32
_CPW = 80
_EP = _NW * _CPW * _CHUNK

_mesh = plsc.VectorSubcoreMesh(core_axis_name="c", subcore_axis_name="s")


@functools.partial(
    pl.kernel,
    out_type=jax.ShapeDtypeStruct((2, _NP, _D), jnp.float32),
    mesh=_mesh,
    scratch_types=[
        pltpu.VMEM_SHARED((_NP, _D), jnp.float32),
        pltpu.VMEM((_CPW // 2, _CHUNK), jnp.int32),
        pltpu.VMEM((_CPW // 2, _CHUNK), jnp.int32),
        pltpu.VMEM((_CHUNK, _D), jnp.float32),
        pltpu.VMEM((_CHUNK, _D), jnp.float32),
        pltpu.SemaphoreType.DMA,
        pltpu.SemaphoreType.DMA,
    ],
)
def _sc_agg(h_hbm, src_hbm, dst_hbm, zeros_hbm, out_hbm,
            agg_sh, src_v, dst_v, rows0, rows1, sem0, sem1):
    c = lax.axis_index("c")
    s = lax.axis_index("s")
    wid = c * 16 + s

    row0 = s * _ROWS_PER_TILE

    @pl.when(jnp.logical_and(c == 0, s < 15))
    def _():
        pltpu.async_copy(h_hbm.at[pl.ds(row0, _ROWS_PER_TILE)],
                         agg_sh.at[pl.ds(row0, _ROWS_PER_TILE)], sem1)

    @pl.when(jnp.logical_and(c == 0, s == 15))
    def _():
        pltpu.sync_copy(zeros_hbm.at[pl.ds(_N, _NP - _N)],
                        agg_sh.at[pl.ds(_N, _NP - _N)])
        pltpu.async_copy(h_hbm.at[pl.ds(row0, _N - 15 * _ROWS_PER_TILE)],
                         agg_sh.at[pl.ds(row0, _N - 15 * _ROWS_PER_TILE)],
                         sem1)

    @pl.when(c == 1)
    def _():
        pltpu.async_copy(zeros_hbm.at[pl.ds(row0, _ROWS_PER_TILE)],
                         agg_sh.at[pl.ds(row0, _ROWS_PER_TILE)], sem1)

    half = _CPW // 2
    for p in range(2):
        base = wid * _CPW + p * half
        pltpu.sync_copy(src_hbm.at[pl.ds(base, half)], src_v)
        pltpu.sync_copy(dst_hbm.at[pl.ds(base, half)], dst_v)
        pltpu.async_copy(h_hbm.at[src_v.at[0]], rows0, sem0)
        if p == 0:
            @pl.when(jnp.logical_or(c == 1, s < 15))
            def _():
                pltpu.make_async_copy(
                    zeros_hbm.at[pl.ds(0, _ROWS_PER_TILE)],
                    agg_sh.at[pl.ds(row0, _ROWS_PER_TILE)], sem1).wait()

            @pl.when(jnp.logical_and(c == 0, s == 15))
            def _():
                pltpu.make_async_copy(
                    zeros_hbm.at[pl.ds(0, _N - 15 * _ROWS_PER_TILE)],
                    agg_sh.at[pl.ds(row0, _N - 15 * _ROWS_PER_TILE)],
                    sem1).wait()

            plsc.subcore_barrier()

        @pl.loop(0, half, step=2)
        def _(j):
            pltpu.async_copy(h_hbm.at[src_v.at[j + 1]], rows1, sem1)
            pltpu.make_async_copy(h_hbm.at[src_v.at[j]], rows0, sem0).wait()
            pltpu.sync_copy(rows0, agg_sh.at[dst_v.at[j]], add=True)

            @pl.when(j + 2 < half)
            def _():
                pltpu.async_copy(h_hbm.at[src_v.at[j + 2]], rows0, sem0)

            pltpu.make_async_copy(h_hbm.at[src_v.at[j + 1]], rows1, sem1).wait()
            pltpu.sync_copy(rows1, agg_sh.at[dst_v.at[j + 1]], add=True)

    plsc.subcore_barrier()

    pltpu.sync_copy(agg_sh.at[pl.ds(row0, _ROWS_PER_TILE)],
                    out_hbm.at[c, pl.ds(row0, _ROWS_PER_TILE)])


def _mlp_body(a_ref, w1_ref, b1_ref, w2_ref, b2_ref, o_ref):
    t = a_ref[0] + a_ref[1]
    t = jnp.dot(t, w1_ref[...], preferred_element_type=jnp.float32) + b1_ref[...]
    t = jnp.maximum(t, 0.0)
    o_ref[...] = (
        jnp.dot(t, w2_ref[...], preferred_element_type=jnp.float32) + b2_ref[...]
    )


_ROW_BLK = 2000


def _mlp(agg, w1, b1, w2, b2):
    grid = (_N // _ROW_BLK,)
    fixed = lambda i: (0, 0)
    return pl.pallas_call(
        _mlp_body,
        grid=grid,
        in_specs=[
            pl.BlockSpec((2, _ROW_BLK, _D), lambda i: (0, i, 0)),
            pl.BlockSpec((_D, _D), fixed),
            pl.BlockSpec((1, _D), fixed),
            pl.BlockSpec((_D, _D), fixed),
            pl.BlockSpec((1, _D), fixed),
        ],
        out_specs=pl.BlockSpec((_ROW_BLK, _D), lambda i: (i, 0)),
        out_shape=jax.ShapeDtypeStruct((_N, _D), jnp.float32),
    )(agg, w1, b1, w2, b2)


def kernel(x, edge_index, W1_0, b1_0, W2_0, b2_0, W1_1, b1_1, W2_1, b2_1,
           W1_2, b1_2, W2_2, b2_2):
    params = [
        (W1_0, b1_0, W2_0, b2_0),
        (W1_1, b1_1, W2_1, b2_1),
        (W1_2, b1_2, W2_2, b2_2),
    ]
    npad = _EP - _E
    pad_iota = jnp.arange(npad, dtype=jnp.int32)
    src2d = jnp.concatenate(
        [edge_index[0], pad_iota % _N]).reshape(-1, _CHUNK)
    dst2d = jnp.concatenate(
        [edge_index[1], _N + pad_iota % (_NP - _N)]).reshape(-1, _CHUNK)
    zeros = jnp.zeros((_NP, _D), jnp.float32)
    h = x
    for (W1, b1, W2, b2) in params:
        agg = _sc_agg(h, src2d, dst2d, zeros)
        h = _mlp(agg, W1, b1.reshape(1, _D), W2, b2.reshape(1, _D))
    return h
